# baseline (speedup 1.0000x reference)
import functools

import jax
import jax.numpy as jnp
from jax import lax
from jax.experimental import pallas as pl
from jax.experimental.pallas import tpu as pltpu
from jax.experimental.pallas import tpu_sc as plsc

N = 10000
E = 320000
H = 128
HP = H // 2
DE = 16
L = 3

NC = 2
NS = 16
LANES = 16

NW = NC * NS
EPT = E // NW
K = 80
NCHUNK = EPT // K
ZCH = 400
NZC = N // ZCH


def _pack_bf16_pairs(z):
    y = lax.bitcast_convert_type(z.astype(jnp.bfloat16), jnp.uint16)
    y = y.astype(jnp.int32)
    return y[:, :HP] | (y[:, HP:] << 16)


def _edge_transform_body(ea_ref, we_ref, be_ref, o_ref):
    e = (jnp.dot(ea_ref[...], we_ref[...],
                 preferred_element_type=jnp.float32) + be_ref[...])
    o_ref[...] = _pack_bf16_pairs(e)


def _edge_transform(edge_attr, We, be):
    BE = 2000
    return pl.pallas_call(
        _edge_transform_body,
        grid=(E // BE,),
        in_specs=[
            pl.BlockSpec((BE, DE), lambda i: (i, 0)),
            pl.BlockSpec((DE, H), lambda i: (0, 0)),
            pl.BlockSpec((H,), lambda i: (0,)),
        ],
        out_specs=pl.BlockSpec((BE, HP), lambda i: (i, 0)),
        out_shape=jax.ShapeDtypeStruct((E, HP), jnp.int32),
    )(edge_attr, We, be)


def _edge_agg_body(h_hbm, e_hbm, src_hbm, dst_hbm, out_hbm,
                   src_v, dst_v, e_v, h_v, agg_sh,
                   sem_e0, sem_e1, sem_g0, sem_g1, sem_s0, sem_s1,
                   sem_d0, sem_d1, sem_r0, sem_r1):
    c = lax.axis_index("c")
    s = lax.axis_index("s")
    wid = c * NS + s
    sem_e = (sem_e0, sem_e1)
    sem_g = (sem_g0, sem_g1)
    sem_s = (sem_s0, sem_s1)
    sem_d = (sem_d0, sem_d1)
    sem_r = (sem_r0, sem_r1)

    zb = h_v.at[0]

    def zrow(i, carry):
        for j in range(H // LANES):
            zb[i, pl.ds(j * LANES, LANES)] = jnp.zeros((LANES,), jnp.float32)
        return carry

    lax.fori_loop(0, K, zrow, 0)
    for r in range((NZC + NS - 1) // NS):
        zc = s + r * NS

        @pl.when(zc < NZC)
        def _zero_chunk():
            for j in range(ZCH // K):
                pltpu.sync_copy(zb, agg_sh.at[pl.ds(zc * ZCH + j * K, K)])

    plsc.subcore_barrier()

    ebase = wid * EPT

    def issue_src(cidx, b):
        pltpu.async_copy(src_hbm.at[pl.ds(ebase + cidx * K, K)],
                         src_v.at[b], sem_r[b])

    def wait_src(b):
        pltpu.make_async_copy(src_hbm.at[pl.ds(ebase, K)],
                              src_v.at[b], sem_r[b]).wait()

    def issue_load(cidx, b):
        eoff = pl.multiple_of(ebase // 2 + cidx * (K // 2), 8)
        pltpu.async_copy(e_hbm.at[pl.ds(eoff, K // 2)],
                         e_v.at[b], sem_e[b])
        pltpu.async_copy(dst_hbm.at[pl.ds(ebase + cidx * K, K)],
                         dst_v.at[b], sem_d[b])
        pltpu.async_copy(h_hbm.at[src_v.at[b]], h_v.at[b], sem_g[b])

    def wait_load(b):
        pltpu.make_async_copy(
            e_hbm.at[pl.ds(pl.multiple_of(ebase // 2, 8), K // 2)],
            e_v.at[b], sem_e[b]).wait()
        pltpu.make_async_copy(dst_hbm.at[pl.ds(ebase, K)],
                              dst_v.at[b], sem_d[b]).wait()
        pltpu.make_async_copy(h_hbm.at[src_v.at[b]],
                              h_v.at[b], sem_g[b]).wait()

    def wait_scatter(b):
        pltpu.make_async_copy(h_v.at[b], agg_sh.at[dst_v.at[b]],
                              sem_s[b]).wait()

    himask = jnp.full((LANES,), -65536, jnp.int32)

    UNROLL = 2

    def compute_scatter(cidx, b):
        eb = e_v.at[b]
        hb = h_v.at[b]

        def rows(r2, rc):
            for u in range(UNROLL):
                r = r2 * UNROLL + u
                for half in range(2):
                    i = 2 * r + half
                    for g in range(HP // LANES):
                        sl = pl.ds(g * LANES, LANES)
                        sh = pl.ds(HP + g * LANES, LANES)
                        ew = eb[r, pl.ds(half * HP + g * LANES, LANES)]
                        el = lax.bitcast_convert_type(ew << 16, jnp.float32)
                        eh = lax.bitcast_convert_type(ew & himask,
                                                      jnp.float32)
                        hb[i, sl] = jnp.maximum(el + hb[i, sl], 0.0)
                        hb[i, sh] = jnp.maximum(eh + hb[i, sh], 0.0)
            return rc

        lax.fori_loop(0, K // 2 // UNROLL, rows, 0)
        pltpu.async_copy(h_v.at[b], agg_sh.at[dst_v.at[b]],
                         sem_s[b], add=True)

    issue_src(0, 0)
    issue_src(1, 1)
    wait_src(0)
    issue_load(0, 0)

    def pair(i, carry):
        c0 = i * 2
        wait_load(0)

        @pl.when(i >= 1)
        def _w():
            wait_scatter(1)

        wait_src(1)
        issue_load(c0 + 1, 1)
        issue_src(c0 + 2, 0)
        compute_scatter(c0, 0)
        wait_load(1)
        wait_scatter(0)
        wait_src(0)
        issue_load(c0 + 2, 0)

        @pl.when(i < NCHUNK // 2 - 1)
        def _s():
            issue_src(c0 + 3, 1)

        compute_scatter(c0 + 1, 1)
        return carry

    lax.fori_loop(0, NCHUNK // 2, pair, 0)
    wait_load(0)
    compute_scatter(NCHUNK - 1, 0)
    wait_scatter(1)
    wait_scatter(0)
    plsc.subcore_barrier()
    for r in range((NZC + NS - 1) // NS):
        wc = s + r * NS

        @pl.when(wc < NZC)
        def _write_chunk():
            pltpu.sync_copy(
                agg_sh.at[pl.ds(wc * ZCH, ZCH)],
                out_hbm.at[c, pl.ds(wc * ZCH, ZCH)],
            )


def _edge_agg(h, e, src, dst):
    mesh = plsc.VectorSubcoreMesh(
        core_axis_name="c", subcore_axis_name="s",
        num_cores=NC, num_subcores=NS,
    )
    fn = functools.partial(
        pl.kernel,
        out_type=jax.ShapeDtypeStruct((NC, N, H), jnp.float32),
        mesh=mesh,
        scratch_types=[
            pltpu.VMEM((2, K), jnp.int32),
            pltpu.VMEM((2, K), jnp.int32),
            pltpu.VMEM((2, K // 2, H), jnp.int32),
            pltpu.VMEM((2, K, H), jnp.float32),
            pltpu.VMEM_SHARED((N, H), jnp.float32),
            pltpu.SemaphoreType.DMA,
            pltpu.SemaphoreType.DMA,
            pltpu.SemaphoreType.DMA,
            pltpu.SemaphoreType.DMA,
            pltpu.SemaphoreType.DMA,
            pltpu.SemaphoreType.DMA,
            pltpu.SemaphoreType.DMA,
            pltpu.SemaphoreType.DMA,
            pltpu.SemaphoreType.DMA,
            pltpu.SemaphoreType.DMA,
        ],
    )(_edge_agg_body)
    return fn(h, e.reshape(E // 2, H), src, dst)


def _mlp_body(h_ref, a0_ref, a1_ref, w1_ref, b1_ref, w2_ref, b2_ref,
              g_ref, bb_ref, o_ref):
    h = h_ref[...]
    z = a0_ref[...] + a1_ref[...] + h
    t = jnp.maximum(
        jnp.dot(z, w1_ref[...], preferred_element_type=jnp.float32)
        + b1_ref[...], 0.0)
    z2 = (jnp.dot(t, w2_ref[...], preferred_element_type=jnp.float32)
          + b2_ref[...])
    mu = jnp.mean(z2, axis=-1, keepdims=True)
    zc = z2 - mu
    var = jnp.mean(zc * zc, axis=-1, keepdims=True)
    zn = zc * lax.rsqrt(var + 1e-5) * g_ref[...] + bb_ref[...]
    o_ref[...] = jnp.maximum(zn, 0.0) + h


def _mlp(h, a0, a1, w1, b1, w2, b2, g, bb):
    BN = 1000
    row = pl.BlockSpec((BN, H), lambda i: (i, 0))
    full = pl.BlockSpec((H, H), lambda i: (0, 0))
    vec = pl.BlockSpec((H,), lambda i: (0,))
    return pl.pallas_call(
        _mlp_body,
        grid=(N // BN,),
        in_specs=[row, row, row, full, vec, full, vec, vec, vec],
        out_specs=row,
        out_shape=jax.ShapeDtypeStruct((N, H), jnp.float32),
    )(h, a0, a1, w1, b1, w2, b2, g, bb)


def kernel(x, batch_index, edge_index, edge_attr, We, be,
           lin1_W, lin1_b, lin2_W, lin2_b, ln_g, ln_b):
    e = _edge_transform(edge_attr, We, be)
    src = edge_index[0].astype(jnp.int32)
    dst = edge_index[1].astype(jnp.int32)
    def layer(h, ws):
        w1, b1, w2, b2, g, bb = ws
        agg = _edge_agg(h, e, src, dst)
        return _mlp(h, agg[0], agg[1], w1, b1, w2, b2, g, bb), None

    h, _ = lax.scan(
        layer, x, (lin1_W, lin1_b, lin2_W, lin2_b, ln_g, ln_b))
    return h


# speedup vs baseline: 1.6800x; 1.6800x over previous
import functools

import jax
import jax.numpy as jnp
from jax import lax
from jax.experimental import pallas as pl
from jax.experimental.pallas import tpu as pltpu
from jax.experimental.pallas import tpu_sc as plsc

N = 10000
E = 320000
H = 128
HP = H // 2
DE = 16
L = 3

NC = 2
NS = 16
LANES = 16

NW = NC * NS
EPT = E // NW
K = 40
NCHUNK = EPT // K
NB = 3
ZCH = 400
NZC = N // ZCH


def _pack_bf16_pairs(z):
    y = lax.bitcast_convert_type(z.astype(jnp.bfloat16), jnp.uint16)
    y = y.astype(jnp.int32)
    return y[:, :HP] | (y[:, HP:] << 16)


def _edge_transform_body(ea_ref, we_ref, be_ref, o_ref):
    e = (jnp.dot(ea_ref[...], we_ref[...],
                 preferred_element_type=jnp.float32) + be_ref[...])
    o_ref[...] = _pack_bf16_pairs(e)


def _edge_transform(edge_attr, We, be):
    BE = 2000
    return pl.pallas_call(
        _edge_transform_body,
        grid=(E // BE,),
        in_specs=[
            pl.BlockSpec((BE, DE), lambda i: (i, 0)),
            pl.BlockSpec((DE, H), lambda i: (0, 0)),
            pl.BlockSpec((H,), lambda i: (0,)),
        ],
        out_specs=pl.BlockSpec((BE, HP), lambda i: (i, 0)),
        out_shape=jax.ShapeDtypeStruct((E, HP), jnp.int32),
    )(edge_attr, We, be)


def _edge_agg_body(h_hbm, e_hbm, src_hbm, dst_hbm, out_hbm,
                   src_v, dst_v, e_v, h_v, agg_sh,
                   sem_e0, sem_e1, sem_e2, sem_g0, sem_g1, sem_g2,
                   sem_s0, sem_s1, sem_s2, sem_d0, sem_d1, sem_d2,
                   sem_r0, sem_r1, sem_r2):
    c = lax.axis_index("c")
    s = lax.axis_index("s")
    wid = c * NS + s
    sem_e = (sem_e0, sem_e1, sem_e2)
    sem_g = (sem_g0, sem_g1, sem_g2)
    sem_s = (sem_s0, sem_s1, sem_s2)
    sem_d = (sem_d0, sem_d1, sem_d2)
    sem_r = (sem_r0, sem_r1, sem_r2)

    zb = h_v.at[0]

    def zrow(i, carry):
        for j in range(H // LANES):
            zb[i, pl.ds(j * LANES, LANES)] = jnp.zeros((LANES,), jnp.float32)
        return carry

    lax.fori_loop(0, K, zrow, 0)
    for r in range((NZC + NS - 1) // NS):
        zc = s + r * NS

        @pl.when(zc < NZC)
        def _zero_chunk():
            for j in range(ZCH // K):
                pltpu.sync_copy(zb, agg_sh.at[pl.ds(zc * ZCH + j * K, K)])

    plsc.subcore_barrier()

    ebase = wid * EPT

    def issue_src(cidx, b):
        pltpu.async_copy(src_hbm.at[pl.ds(ebase + cidx * K, K)],
                         src_v.at[b], sem_r[b])

    def wait_src(b):
        pltpu.make_async_copy(src_hbm.at[pl.ds(ebase, K)],
                              src_v.at[b], sem_r[b]).wait()

    def issue_load(cidx, b):
        pltpu.async_copy(e_hbm.at[pl.ds(ebase + cidx * K, K)],
                         e_v.at[b], sem_e[b])
        pltpu.async_copy(dst_hbm.at[pl.ds(ebase + cidx * K, K)],
                         dst_v.at[b], sem_d[b])
        pltpu.async_copy(h_hbm.at[src_v.at[b]], h_v.at[b], sem_g[b])

    def wait_load(b):
        pltpu.make_async_copy(e_hbm.at[pl.ds(ebase, K)],
                              e_v.at[b], sem_e[b]).wait()
        pltpu.make_async_copy(dst_hbm.at[pl.ds(ebase, K)],
                              dst_v.at[b], sem_d[b]).wait()
        pltpu.make_async_copy(h_hbm.at[src_v.at[b]],
                              h_v.at[b], sem_g[b]).wait()

    def wait_scatter(b):
        pltpu.make_async_copy(h_v.at[b], agg_sh.at[dst_v.at[b]],
                              sem_s[b]).wait()

    himask = jnp.full((LANES,), -65536, jnp.int32)

    UNROLL = 4

    def compute_scatter(cidx, b):
        eb = e_v.at[b]
        hb = h_v.at[b]

        def rows(i4, rc):
            for u in range(UNROLL):
                i = i4 * UNROLL + u
                for g in range(HP // LANES):
                    sl = pl.ds(g * LANES, LANES)
                    sh = pl.ds(HP + g * LANES, LANES)
                    ew = eb[i, sl]
                    el = lax.bitcast_convert_type(ew << 16, jnp.float32)
                    eh = lax.bitcast_convert_type(ew & himask, jnp.float32)
                    hb[i, sl] = jnp.maximum(el + hb[i, sl], 0.0)
                    hb[i, sh] = jnp.maximum(eh + hb[i, sh], 0.0)
            return rc

        lax.fori_loop(0, K // UNROLL, rows, 0)
        pltpu.async_copy(h_v.at[b], agg_sh.at[dst_v.at[b]],
                         sem_s[b], add=True)

    issue_src(0, 0)
    issue_src(1, 1)
    issue_src(2, 2)
    wait_src(0)
    issue_load(0, 0)
    wait_src(1)
    issue_load(1, 1)

    NTRIP = (NCHUNK - 4) // NB

    def trip(i, carry):
        c0 = i * NB
        for b in range(NB):
            c = c0 + b
            bn = (b + 2) % NB
            if b == 0:
                @pl.when(i >= 1)
                def _w():
                    wait_scatter(bn)
            else:
                wait_scatter(bn)
            wait_load(b)
            wait_src(bn)
            issue_load(c + 2, bn)
            issue_src(c + 3, b)
            compute_scatter(c, b)
        return carry

    lax.fori_loop(0, NTRIP, trip, 0)
    for c in range(NTRIP * NB, NCHUNK):
        b = c % NB
        bn = (b + 2) % NB
        wait_scatter(bn)
        wait_load(b)
        if c + 2 < NCHUNK:
            wait_src(bn)
            issue_load(c + 2, bn)
        if c + 3 < NCHUNK:
            issue_src(c + 3, b)
        compute_scatter(c, b)
    wait_scatter((NCHUNK - 1) % NB)
    plsc.subcore_barrier()
    for r in range((NZC + NS - 1) // NS):
        wc = s + r * NS

        @pl.when(wc < NZC)
        def _write_chunk():
            pltpu.sync_copy(
                agg_sh.at[pl.ds(wc * ZCH, ZCH)],
                out_hbm.at[c, pl.ds(wc * ZCH, ZCH)],
            )


def _edge_agg(h, e, src, dst):
    mesh = plsc.VectorSubcoreMesh(
        core_axis_name="c", subcore_axis_name="s",
        num_cores=NC, num_subcores=NS,
    )
    fn = functools.partial(
        pl.kernel,
        out_type=jax.ShapeDtypeStruct((NC, N, H), jnp.float32),
        mesh=mesh,
        scratch_types=[
            pltpu.VMEM((NB, K), jnp.int32),
            pltpu.VMEM((NB, K), jnp.int32),
            pltpu.VMEM((NB, K, HP), jnp.int32),
            pltpu.VMEM((NB, K, H), jnp.float32),
            pltpu.VMEM_SHARED((N, H), jnp.float32),
        ] + [pltpu.SemaphoreType.DMA] * 15,
    )(_edge_agg_body)
    return fn(h, e, src, dst)


def _mlp_body(h_ref, a0_ref, a1_ref, w1_ref, b1_ref, w2_ref, b2_ref,
              g_ref, bb_ref, o_ref):
    h = h_ref[...]
    z = a0_ref[...] + a1_ref[...] + h
    t = jnp.maximum(
        jnp.dot(z, w1_ref[...], preferred_element_type=jnp.float32)
        + b1_ref[...], 0.0)
    z2 = (jnp.dot(t, w2_ref[...], preferred_element_type=jnp.float32)
          + b2_ref[...])
    mu = jnp.mean(z2, axis=-1, keepdims=True)
    zc = z2 - mu
    var = jnp.mean(zc * zc, axis=-1, keepdims=True)
    zn = zc * lax.rsqrt(var + 1e-5) * g_ref[...] + bb_ref[...]
    o_ref[...] = jnp.maximum(zn, 0.0) + h


def _mlp(h, a0, a1, w1, b1, w2, b2, g, bb):
    BN = 1000
    row = pl.BlockSpec((BN, H), lambda i: (i, 0))
    full = pl.BlockSpec((H, H), lambda i: (0, 0))
    vec = pl.BlockSpec((H,), lambda i: (0,))
    return pl.pallas_call(
        _mlp_body,
        grid=(N // BN,),
        in_specs=[row, row, row, full, vec, full, vec, vec, vec],
        out_specs=row,
        out_shape=jax.ShapeDtypeStruct((N, H), jnp.float32),
    )(h, a0, a1, w1, b1, w2, b2, g, bb)


def kernel(x, batch_index, edge_index, edge_attr, We, be,
           lin1_W, lin1_b, lin2_W, lin2_b, ln_g, ln_b):
    e = _edge_transform(edge_attr, We, be)
    src = edge_index[0].astype(jnp.int32)
    dst = edge_index[1].astype(jnp.int32)
    def layer(h, ws):
        w1, b1, w2, b2, g, bb = ws
        agg = _edge_agg(h, e, src, dst)
        return _mlp(h, agg[0], agg[1], w1, b1, w2, b2, g, bb), None

    h, _ = lax.scan(
        layer, x, (lin1_W, lin1_b, lin2_W, lin2_b, ln_g, ln_b))
    return h


# speedup vs baseline: 1.6819x; 1.0011x over previous
import functools

import jax
import jax.numpy as jnp
from jax import lax
from jax.experimental import pallas as pl
from jax.experimental.pallas import tpu as pltpu
from jax.experimental.pallas import tpu_sc as plsc

N = 10000
E = 320000
H = 128
HP = H // 2
DE = 16
L = 3

NC = 2
NS = 16
LANES = 16

NW = NC * NS
EPT = E // NW
K = 40
NCHUNK = EPT // K
NB = 3
ZCH = 400
NZC = N // ZCH


def _pack_bf16_pairs(z):
    y = lax.bitcast_convert_type(z.astype(jnp.bfloat16), jnp.uint16)
    y = y.astype(jnp.int32)
    return y[:, :HP] | (y[:, HP:] << 16)


def _edge_transform_body(ea_ref, we_ref, be_ref, o_ref):
    e = (jnp.dot(ea_ref[...], we_ref[...],
                 preferred_element_type=jnp.float32) + be_ref[...])
    o_ref[...] = _pack_bf16_pairs(e)


def _edge_transform(edge_attr, We, be):
    BE = 2000
    return pl.pallas_call(
        _edge_transform_body,
        grid=(E // BE,),
        in_specs=[
            pl.BlockSpec((BE, DE), lambda i: (i, 0)),
            pl.BlockSpec((DE, H), lambda i: (0, 0)),
            pl.BlockSpec((H,), lambda i: (0,)),
        ],
        out_specs=pl.BlockSpec((BE, HP), lambda i: (i, 0)),
        out_shape=jax.ShapeDtypeStruct((E, HP), jnp.int32),
    )(edge_attr, We, be)


def _edge_agg_body(h_hbm, e_hbm, src_hbm, dst_hbm, out_hbm,
                   src_v, dst_v, e_v, h_v, agg_sh,
                   sem_e0, sem_e1, sem_e2, sem_g0, sem_g1, sem_g2,
                   sem_s0, sem_s1, sem_s2, sem_d0, sem_d1, sem_d2,
                   sem_r0, sem_r1, sem_r2):
    c = lax.axis_index("c")
    s = lax.axis_index("s")
    wid = c * NS + s
    sem_e = (sem_e0, sem_e1, sem_e2)
    sem_g = (sem_g0, sem_g1, sem_g2)
    sem_s = (sem_s0, sem_s1, sem_s2)
    sem_d = (sem_d0, sem_d1, sem_d2)
    sem_r = (sem_r0, sem_r1, sem_r2)

    zb = h_v.at[0]

    def zrow(i, carry):
        for j in range(H // LANES):
            zb[i, pl.ds(j * LANES, LANES)] = jnp.zeros((LANES,), jnp.float32)
        return carry

    lax.fori_loop(0, K, zrow, 0)
    for r in range((NZC + NS - 1) // NS):
        zc = s + r * NS

        @pl.when(zc < NZC)
        def _zero_chunk():
            for j in range(ZCH // K):
                pltpu.sync_copy(zb, agg_sh.at[pl.ds(zc * ZCH + j * K, K)])

    plsc.subcore_barrier()

    ebase = wid * EPT

    def issue_src(cidx, b):
        pltpu.async_copy(src_hbm.at[pl.ds(ebase + cidx * K, K)],
                         src_v.at[b], sem_r[b])

    def wait_src(b):
        pltpu.make_async_copy(src_hbm.at[pl.ds(ebase, K)],
                              src_v.at[b], sem_r[b]).wait()

    def issue_load(cidx, b):
        pltpu.async_copy(e_hbm.at[pl.ds(ebase + cidx * K, K)],
                         e_v.at[b], sem_e[b])
        pltpu.async_copy(dst_hbm.at[pl.ds(ebase + cidx * K, K)],
                         dst_v.at[b], sem_d[b])
        pltpu.async_copy(h_hbm.at[src_v.at[b]], h_v.at[b], sem_g[b])

    def wait_load(b):
        pltpu.make_async_copy(e_hbm.at[pl.ds(ebase, K)],
                              e_v.at[b], sem_e[b]).wait()
        pltpu.make_async_copy(dst_hbm.at[pl.ds(ebase, K)],
                              dst_v.at[b], sem_d[b]).wait()
        pltpu.make_async_copy(h_hbm.at[src_v.at[b]],
                              h_v.at[b], sem_g[b]).wait()

    def wait_scatter(b):
        pltpu.make_async_copy(h_v.at[b], agg_sh.at[dst_v.at[b]],
                              sem_s[b]).wait()

    himask = jnp.full((LANES,), -65536, jnp.int32)

    UNROLL = 4

    def compute_scatter(cidx, b):
        eb = e_v.at[b]
        hb = h_v.at[b]

        def rows(i4, rc):
            for u in range(UNROLL):
                i = i4 * UNROLL + u
                for g in range(HP // LANES):
                    sl = pl.ds(g * LANES, LANES)
                    sh = pl.ds(HP + g * LANES, LANES)
                    ew = eb[i, sl]
                    el = lax.bitcast_convert_type(ew << 16, jnp.float32)
                    eh = lax.bitcast_convert_type(ew & himask, jnp.float32)
                    hb[i, sl] = jnp.maximum(el + hb[i, sl], 0.0)
                    hb[i, sh] = jnp.maximum(eh + hb[i, sh], 0.0)
            return rc

        lax.fori_loop(0, K // UNROLL, rows, 0)
        pltpu.async_copy(h_v.at[b], agg_sh.at[dst_v.at[b]],
                         sem_s[b], add=True)

    issue_src(0, 0)
    issue_src(1, 1)
    issue_src(2, 2)
    wait_src(0)
    issue_load(0, 0)
    wait_src(1)
    issue_load(1, 1)

    NTRIP = (NCHUNK - 4) // NB

    def trip(i, carry):
        c0 = i * NB
        for b in range(NB):
            c = c0 + b
            bn = (b + 2) % NB
            if b == 0:
                @pl.when(i >= 1)
                def _w():
                    wait_scatter(bn)
            else:
                wait_scatter(bn)
            wait_load(b)
            wait_src(bn)
            issue_load(c + 2, bn)
            issue_src(c + 3, b)
            compute_scatter(c, b)
        return carry

    lax.fori_loop(0, NTRIP, trip, 0)
    for ct in range(NTRIP * NB, NCHUNK):
        b = ct % NB
        bn = (b + 2) % NB
        wait_scatter(bn)
        wait_load(b)
        if ct + 2 < NCHUNK:
            wait_src(bn)
            issue_load(ct + 2, bn)
        if ct + 3 < NCHUNK:
            issue_src(ct + 3, b)
        compute_scatter(ct, b)
    wait_scatter((NCHUNK - 1) % NB)
    plsc.subcore_barrier()
    for r in range((NZC + NS - 1) // NS):
        wc = s + r * NS

        @pl.when(wc < NZC)
        def _write_chunk():
            pltpu.sync_copy(
                agg_sh.at[pl.ds(wc * ZCH, ZCH)],
                out_hbm.at[c, pl.ds(wc * ZCH, ZCH)],
            )


def _edge_agg(h, e, src, dst):
    mesh = plsc.VectorSubcoreMesh(
        core_axis_name="c", subcore_axis_name="s",
        num_cores=NC, num_subcores=NS,
    )
    fn = functools.partial(
        pl.kernel,
        out_type=jax.ShapeDtypeStruct((NC, N, H), jnp.float32),
        mesh=mesh,
        scratch_types=[
            pltpu.VMEM((NB, K), jnp.int32),
            pltpu.VMEM((NB, K), jnp.int32),
            pltpu.VMEM((NB, K, HP), jnp.int32),
            pltpu.VMEM((NB, K, H), jnp.float32),
            pltpu.VMEM_SHARED((N, H), jnp.float32),
        ] + [pltpu.SemaphoreType.DMA] * 15,
    )(_edge_agg_body)
    return fn(h, e, src, dst)


def _mlp_body(h_ref, a0_ref, a1_ref, w1_ref, b1_ref, w2_ref, b2_ref,
              g_ref, bb_ref, o_ref):
    h = h_ref[...]
    z = a0_ref[...] + a1_ref[...] + h
    t = jnp.maximum(
        jnp.dot(z, w1_ref[...], preferred_element_type=jnp.float32)
        + b1_ref[...], 0.0)
    z2 = (jnp.dot(t, w2_ref[...], preferred_element_type=jnp.float32)
          + b2_ref[...])
    mu = jnp.mean(z2, axis=-1, keepdims=True)
    zc = z2 - mu
    var = jnp.mean(zc * zc, axis=-1, keepdims=True)
    zn = zc * lax.rsqrt(var + 1e-5) * g_ref[...] + bb_ref[...]
    o_ref[...] = jnp.maximum(zn, 0.0) + h


def _mlp(h, a0, a1, w1, b1, w2, b2, g, bb):
    BN = 1000
    row = pl.BlockSpec((BN, H), lambda i: (i, 0))
    full = pl.BlockSpec((H, H), lambda i: (0, 0))
    vec = pl.BlockSpec((H,), lambda i: (0,))
    return pl.pallas_call(
        _mlp_body,
        grid=(N // BN,),
        in_specs=[row, row, row, full, vec, full, vec, vec, vec],
        out_specs=row,
        out_shape=jax.ShapeDtypeStruct((N, H), jnp.float32),
    )(h, a0, a1, w1, b1, w2, b2, g, bb)


def kernel(x, batch_index, edge_index, edge_attr, We, be,
           lin1_W, lin1_b, lin2_W, lin2_b, ln_g, ln_b):
    e = _edge_transform(edge_attr, We, be)
    src = edge_index[0].astype(jnp.int32)
    dst = edge_index[1].astype(jnp.int32)
    def layer(h, ws):
        w1, b1, w2, b2, g, bb = ws
        agg = _edge_agg(h, e, src, dst)
        return _mlp(h, agg[0], agg[1], w1, b1, w2, b2, g, bb), None

    h, _ = lax.scan(
        layer, x, (lin1_W, lin1_b, lin2_W, lin2_b, ln_g, ln_b))
    return h
